# baseline (speedup 1.0000x reference)
import jax
import jax.numpy as jnp
from jax.experimental import pallas as pl


def _add_body(x_ref, pos_ref, o_ref):
    o_ref[...] = x_ref[...] + pos_ref[...][None]


def kernel(patch, pos_table):
    B, P, D = patch.shape
    xt = jnp.transpose(patch, (0, 2, 1))
    post = jnp.transpose(pos_table, (1, 0))
    BB = 64
    PP = 256
    out_t = pl.pallas_call(
        _add_body,
        grid=(B // BB, P // PP),
        in_specs=[
            pl.BlockSpec((BB, D, PP), lambda i, j: (i, 0, j)),
            pl.BlockSpec((D, PP), lambda i, j: (0, j)),
        ],
        out_specs=pl.BlockSpec((BB, D, PP), lambda i, j: (i, 0, j)),
        out_shape=jax.ShapeDtypeStruct((B, D, P), jnp.float32),
    )(xt, post)
    return jnp.transpose(out_t, (0, 2, 1))
